# Initial kernel scaffold; baseline (speedup 1.0000x reference)
import jax
import jax.numpy as jnp
from jax.experimental import pallas as pl


def kernel(x, edge_index, W):
    raise NotImplementedError("write your pallas kernel here")


# speedup vs baseline: 10.1860x; 10.1860x over previous
import functools

import jax
import jax.numpy as jnp
from jax import lax
from jax.experimental import pallas as pl
from jax.experimental.pallas import tpu as pltpu
from jax.experimental.pallas import tpu_sc as plsc

N = 10000
NP = 10240
E = 160000
EP = 163840
D = 256
DH = 128
NC, NS = 2, 16
CH = 128
DEG_CHUNKS = EP // (NC * NS) // CH
MAIN_CHUNKS = EP // (NS * CH)
RPT = NP // NS
BLK = 2048

_mesh = plsc.VectorSubcoreMesh(core_axis_name="c", subcore_axis_name="s")


@functools.partial(
    pl.kernel,
    out_type=jax.ShapeDtypeStruct((NC, NP), jnp.float32),
    mesh=_mesh,
    scratch_types=[
        pltpu.VMEM((DEG_CHUNKS, CH), jnp.int32),
        pltpu.VMEM((RPT,), jnp.float32),
        pltpu.VMEM((CH,), jnp.float32),
        pltpu.VMEM_SHARED((NP,), jnp.float32),
    ],
)
def _deg_kernel(rows_hbm, out_hbm, idx_v, zbuf, ones_v, deg_sh):
    c = lax.axis_index("c")
    s = lax.axis_index("s")

    def zfill(i, _):
        zbuf[pl.ds(i * 16, 16)] = jnp.zeros((16,), jnp.float32)
        return 0

    lax.fori_loop(0, RPT // 16, zfill, 0)
    for u in range(CH // 16):
        ones_v[pl.ds(u * 16, 16)] = jnp.ones((16,), jnp.float32)

    pltpu.sync_copy(zbuf, deg_sh.at[pl.ds(s * RPT, RPT)])
    pltpu.sync_copy(rows_hbm.at[c, s], idx_v)
    plsc.subcore_barrier()

    def body(j, _):
        pltpu.sync_copy(ones_v, deg_sh.at[idx_v.at[j]], add=True)
        return 0

    lax.fori_loop(0, DEG_CHUNKS, body, 0)
    plsc.subcore_barrier()
    pltpu.sync_copy(deg_sh.at[pl.ds(s * RPT, RPT)],
                    out_hbm.at[c].at[pl.ds(s * RPT, RPT)])


@functools.partial(
    pl.kernel,
    out_type=jax.ShapeDtypeStruct((NC, NP, DH), jnp.float32),
    mesh=_mesh,
    scratch_types=[
        pltpu.VMEM((MAIN_CHUNKS, CH), jnp.int32),
        pltpu.VMEM((MAIN_CHUNKS, CH), jnp.int32),
        pltpu.VMEM((CH, DH), jnp.float32),
        pltpu.VMEM_SHARED((NP, DH), jnp.float32),
        pltpu.SemaphoreType.DMA,
    ],
)
def _edge_kernel(hd2_hbm, gidx_hbm, cidx_hbm, out_hbm,
                 gidx_v, cidx_v, rows_v, acc_sh, sem):
    c = lax.axis_index("c")
    s = lax.axis_index("s")

    def zfill(i, _):
        for u in range(DH // 16):
            rows_v[i, pl.ds(u * 16, 16)] = jnp.zeros((16,), jnp.float32)
        return 0

    lax.fori_loop(0, CH, zfill, 0)
    for b in range(RPT // CH):
        pltpu.sync_copy(rows_v, acc_sh.at[pl.ds(s * RPT + b * CH, CH)])

    pltpu.sync_copy(gidx_hbm.at[c, s], gidx_v)
    pltpu.sync_copy(cidx_hbm.at[s], cidx_v)
    plsc.subcore_barrier()

    def body(j, _):
        pltpu.async_copy(hd2_hbm.at[gidx_v.at[j]], rows_v, sem).wait()
        pltpu.sync_copy(rows_v, acc_sh.at[cidx_v.at[j]], add=True)
        return 0

    lax.fori_loop(0, MAIN_CHUNKS, body, 0)
    plsc.subcore_barrier()
    pltpu.sync_copy(acc_sh.at[pl.ds(s * RPT, RPT)],
                    out_hbm.at[c].at[pl.ds(s * RPT, RPT)])


@functools.partial(
    pl.pallas_call,
    grid=(NP // BLK,),
    in_specs=[
        pl.BlockSpec((BLK, D), lambda i: (i, 0)),
        pl.BlockSpec((D, D), lambda i: (0, 0)),
        pl.BlockSpec((NC, BLK), lambda i: (0, i)),
    ],
    out_specs=[
        pl.BlockSpec((BLK, D), lambda i: (i, 0)),
        pl.BlockSpec((BLK,), lambda i: (i,)),
    ],
    out_shape=[
        jax.ShapeDtypeStruct((NP, D), jnp.float32),
        jax.ShapeDtypeStruct((NP,), jnp.float32),
    ],
)
def _matmul_scale(x_ref, w_ref, degp_ref, hd_ref, d_ref):
    deg = 1.0 + degp_ref[0, :] + degp_ref[1, :]
    d = lax.rsqrt(deg)
    h = jnp.dot(x_ref[...], w_ref[...], preferred_element_type=jnp.float32)
    hd_ref[...] = h * d[:, None]
    d_ref[...] = d


@functools.partial(
    pl.pallas_call,
    grid=(NP // BLK,),
    in_specs=[
        pl.BlockSpec((NC, BLK, DH), lambda i: (0, i, 0)),
        pl.BlockSpec((BLK, D), lambda i: (i, 0)),
        pl.BlockSpec((BLK,), lambda i: (i,)),
    ],
    out_specs=pl.BlockSpec((BLK, D), lambda i: (i, 0)),
    out_shape=jax.ShapeDtypeStruct((NP, D), jnp.float32),
)
def _finalize(acc_ref, hd_ref, d_ref, out_ref):
    d = d_ref[...][:, None]
    out_ref[:, :DH] = d * (acc_ref[0] + hd_ref[:, :DH])
    out_ref[:, DH:] = d * (acc_ref[1] + hd_ref[:, DH:])


def kernel(x, edge_index, W):
    row = edge_index[0]
    col = edge_index[1]

    trash = jnp.full((EP - E,), NP - 1, jnp.int32)
    rows_deg = jnp.concatenate([row, trash]).reshape(NC, NS, DEG_CHUNKS, CH)
    degp = _deg_kernel(rows_deg)

    xp = jnp.pad(x, ((0, NP - N), (0, 0)))
    hd, d = _matmul_scale(xp, W, degp)

    rowp = jnp.concatenate([row, jnp.zeros((EP - E,), jnp.int32)])
    g0 = (rowp * 2).reshape(NS, MAIN_CHUNKS, CH)
    gidx = jnp.stack([g0, g0 + 1])
    cidx = jnp.concatenate([col, trash]).reshape(NS, MAIN_CHUNKS, CH)
    hd2 = hd.reshape(NP * 2, DH)
    acc = _edge_kernel(hd2, gidx, cidx)

    out = _finalize(acc, hd, d)
    return out[:N]


# speedup vs baseline: 11.8078x; 1.1592x over previous
import functools

import jax
import jax.numpy as jnp
from jax import lax
from jax.experimental import pallas as pl
from jax.experimental.pallas import tpu as pltpu
from jax.experimental.pallas import tpu_sc as plsc

N = 10000
NP = 10240
E = 160000
EP = 163840
D = 256
DH = 128
NC, NS = 2, 16
CH = 128
DEG_CHUNKS = EP // (NC * NS) // CH
MAIN_CHUNKS = EP // (NS * CH)
RPT = NP // NS
BLK = 2048

_mesh = plsc.VectorSubcoreMesh(core_axis_name="c", subcore_axis_name="s")


@functools.partial(
    pl.kernel,
    out_type=jax.ShapeDtypeStruct((NC, NP), jnp.float32),
    mesh=_mesh,
    scratch_types=[
        pltpu.VMEM((DEG_CHUNKS, CH), jnp.int32),
        pltpu.VMEM((RPT,), jnp.float32),
        pltpu.VMEM((CH,), jnp.float32),
        pltpu.VMEM_SHARED((NP,), jnp.float32),
    ],
)
def _deg_kernel(rows_hbm, out_hbm, idx_v, zbuf, ones_v, deg_sh):
    c = lax.axis_index("c")
    s = lax.axis_index("s")

    def zfill(i, _):
        zbuf[pl.ds(i * 16, 16)] = jnp.zeros((16,), jnp.float32)
        return 0

    lax.fori_loop(0, RPT // 16, zfill, 0)
    for u in range(CH // 16):
        ones_v[pl.ds(u * 16, 16)] = jnp.ones((16,), jnp.float32)

    pltpu.sync_copy(zbuf, deg_sh.at[pl.ds(s * RPT, RPT)])
    pltpu.sync_copy(rows_hbm.at[c, s], idx_v)
    plsc.subcore_barrier()

    def body(j, _):
        pltpu.sync_copy(ones_v, deg_sh.at[idx_v.at[j]], add=True)
        return 0

    lax.fori_loop(0, DEG_CHUNKS, body, 0)
    plsc.subcore_barrier()
    pltpu.sync_copy(deg_sh.at[pl.ds(s * RPT, RPT)],
                    out_hbm.at[c].at[pl.ds(s * RPT, RPT)])


@functools.partial(
    pl.kernel,
    out_type=jax.ShapeDtypeStruct((NC, NP, DH), jnp.float32),
    mesh=_mesh,
    scratch_types=[
        pltpu.VMEM((MAIN_CHUNKS // 2, CH), jnp.int32),
        pltpu.VMEM((MAIN_CHUNKS // 2, CH), jnp.int32),
        pltpu.VMEM((CH, DH), jnp.float32),
        pltpu.VMEM((CH, DH), jnp.float32),
        pltpu.VMEM_SHARED((NP, DH), jnp.float32),
        pltpu.SemaphoreType.DMA,
        pltpu.SemaphoreType.DMA,
    ],
)
def _edge_kernel(hd2_hbm, gidx_hbm, cidx_hbm, out_hbm,
                 gidx_v, cidx_v, rows_v0, rows_v1, acc_sh, sem0, sem1):
    c = lax.axis_index("c")
    s = lax.axis_index("s")

    def zfill(i, _):
        for u in range(DH // 16):
            rows_v0[i, pl.ds(u * 16, 16)] = jnp.zeros((16,), jnp.float32)
        return 0

    lax.fori_loop(0, CH, zfill, 0)
    for b in range(RPT // CH):
        pltpu.sync_copy(rows_v0, acc_sh.at[pl.ds(s * RPT + b * CH, CH)])

    plsc.subcore_barrier()

    HP = MAIN_CHUNKS // 2
    for phase in range(2):
        pltpu.sync_copy(gidx_hbm.at[c, s].at[pl.ds(phase * HP, HP)], gidx_v)
        pltpu.sync_copy(cidx_hbm.at[s].at[pl.ds(phase * HP, HP)], cidx_v)
        pltpu.async_copy(hd2_hbm.at[gidx_v.at[0]], rows_v0, sem0)

        def body(j, _):
            a = 2 * j
            b = a + 1
            pltpu.make_async_copy(
                hd2_hbm.at[gidx_v.at[a]], rows_v0, sem0).wait()
            pltpu.async_copy(hd2_hbm.at[gidx_v.at[b]], rows_v1, sem1)
            pltpu.sync_copy(rows_v0, acc_sh.at[cidx_v.at[a]], add=True)

            @pl.when(j < HP // 2 - 1)
            def _():
                pltpu.async_copy(hd2_hbm.at[gidx_v.at[a + 2]], rows_v0, sem0)

            pltpu.make_async_copy(
                hd2_hbm.at[gidx_v.at[b]], rows_v1, sem1).wait()
            pltpu.sync_copy(rows_v1, acc_sh.at[cidx_v.at[b]], add=True)
            return 0

        lax.fori_loop(0, HP // 2, body, 0)
    plsc.subcore_barrier()
    pltpu.sync_copy(acc_sh.at[pl.ds(s * RPT, RPT)],
                    out_hbm.at[c].at[pl.ds(s * RPT, RPT)])


@functools.partial(
    pl.pallas_call,
    grid=(NP // BLK,),
    in_specs=[
        pl.BlockSpec((BLK, D), lambda i: (i, 0)),
        pl.BlockSpec((D, D), lambda i: (0, 0)),
        pl.BlockSpec((NC, BLK), lambda i: (0, i)),
    ],
    out_specs=[
        pl.BlockSpec((BLK, D), lambda i: (i, 0)),
        pl.BlockSpec((BLK,), lambda i: (i,)),
    ],
    out_shape=[
        jax.ShapeDtypeStruct((NP, D), jnp.float32),
        jax.ShapeDtypeStruct((NP,), jnp.float32),
    ],
)
def _matmul_scale(x_ref, w_ref, degp_ref, hd_ref, d_ref):
    deg = 1.0 + degp_ref[0, :] + degp_ref[1, :]
    d = lax.rsqrt(deg)
    h = jnp.dot(x_ref[...], w_ref[...], preferred_element_type=jnp.float32)
    hd_ref[...] = h * d[:, None]
    d_ref[...] = d


@functools.partial(
    pl.pallas_call,
    grid=(NP // BLK,),
    in_specs=[
        pl.BlockSpec((NC, BLK, DH), lambda i: (0, i, 0)),
        pl.BlockSpec((BLK, D), lambda i: (i, 0)),
        pl.BlockSpec((BLK,), lambda i: (i,)),
    ],
    out_specs=pl.BlockSpec((BLK, D), lambda i: (i, 0)),
    out_shape=jax.ShapeDtypeStruct((NP, D), jnp.float32),
)
def _finalize(acc_ref, hd_ref, d_ref, out_ref):
    d = d_ref[...][:, None]
    out_ref[:, :DH] = d * (acc_ref[0] + hd_ref[:, :DH])
    out_ref[:, DH:] = d * (acc_ref[1] + hd_ref[:, DH:])


def kernel(x, edge_index, W):
    row = edge_index[0]
    col = edge_index[1]

    trash = jnp.full((EP - E,), NP - 1, jnp.int32)
    rows_deg = jnp.concatenate([row, trash]).reshape(NC, NS, DEG_CHUNKS, CH)
    degp = _deg_kernel(rows_deg)

    xp = jnp.pad(x, ((0, NP - N), (0, 0)))
    hd, d = _matmul_scale(xp, W, degp)

    rowp = jnp.concatenate([row, jnp.zeros((EP - E,), jnp.int32)])
    g0 = (rowp * 2).reshape(NS, MAIN_CHUNKS, CH)
    gidx = jnp.stack([g0, g0 + 1])
    cidx = jnp.concatenate([col, trash]).reshape(NS, MAIN_CHUNKS, CH)
    hd2 = hd.reshape(NP * 2, DH)
    acc = _edge_kernel(hd2, gidx, cidx)

    out = _finalize(acc, hd, d)
    return out[:N]
